# baseline (speedup 1.0000x reference)
import functools

import jax
import jax.numpy as jnp
from jax import lax
from jax.experimental import pallas as pl
from jax.experimental.pallas import tpu as pltpu
from jax.experimental.pallas import tpu_sc as plsc

N = 32768
D = 256
H = 128
M = 16
BLK = 8192

_SC_INFO = plsc.get_sparse_core_info()
NC = _SC_INFO.num_cores
NS = _SC_INFO.num_subcores
NW = NC * NS
CHUNK = N // NW
L = 16


def _mlp_body(x_ref, w1_ref, b1_ref, w2_ref, b2_ref, y_ref):
    x = x_ref[...].astype(jnp.bfloat16)
    h = jnp.dot(x, w1_ref[...].astype(jnp.bfloat16),
                preferred_element_type=jnp.float32)
    h = h + b1_ref[...]
    h = h * jax.nn.sigmoid(h)
    y = jnp.dot(h, w2_ref[...], preferred_element_type=jnp.float32)
    y_ref[...] = y + b2_ref[...]


def _mlp(x, w1, b1, w2, b2):
    return pl.pallas_call(
        _mlp_body,
        grid=(N // BLK,),
        in_specs=[
            pl.BlockSpec((BLK, D), lambda i: (i, 0)),
            pl.BlockSpec((D, H), lambda i: (0, 0)),
            pl.BlockSpec((1, H), lambda i: (0, 0)),
            pl.BlockSpec((H, 1), lambda i: (0, 0)),
            pl.BlockSpec((1, 1), lambda i: (0, 0)),
        ],
        out_specs=pl.BlockSpec((BLK, 1), lambda i: (i, 0)),
        out_shape=jax.ShapeDtypeStruct((N, 1), jnp.float32),
    )(x, w1, b1.reshape(1, H), w2, b2.reshape(1, 1))


@functools.partial(
    pl.kernel,
    mesh=plsc.VectorSubcoreMesh(core_axis_name="c", subcore_axis_name="s"),
    out_type=jax.ShapeDtypeStruct((NC, 128), jnp.float32),
    scratch_types=[
        pltpu.VMEM((CHUNK,), jnp.float32),
        pltpu.VMEM((CHUNK,), jnp.int32),
        pltpu.VMEM((128,), jnp.float32),
        pltpu.VMEM((NS, 128), jnp.float32),
        pltpu.VMEM_SHARED((NS, 128), jnp.float32),
    ],
    compiler_params=pltpu.CompilerParams(needs_layout_passes=False),
)
def _sc_segsum(y_hbm, idx_hbm, out_hbm, y_v, idx_v, acc_v, red_v, shared):
    c = lax.axis_index("c")
    s = lax.axis_index("s")
    base = (c * NS + s) * CHUNK
    pltpu.sync_copy(y_hbm.at[pl.ds(base, CHUNK)], y_v)
    pltpu.sync_copy(idx_hbm.at[pl.ds(base, CHUNK)], idx_v)
    for j in range(128 // L):
        acc_v[pl.ds(j * L, L)] = jnp.zeros((L,), jnp.float32)

    def body(j, carry):
        off = j * L
        v = y_v[pl.ds(off, L)]
        ix = idx_v[pl.ds(off, L)]
        plsc.addupdate_scatter(acc_v, [ix], v)
        return carry

    lax.fori_loop(0, CHUNK // L, body, 0)

    pltpu.sync_copy(acc_v, shared.at[s])
    plsc.subcore_barrier()

    @pl.when(s == 0)
    def _reduce():
        pltpu.sync_copy(shared, red_v)
        total = red_v[0, pl.ds(0, M)]
        for j in range(1, NS):
            total = total + red_v[j, pl.ds(0, M)]
        acc_v[pl.ds(0, M)] = total
        pltpu.sync_copy(acc_v, out_hbm.at[c])


def kernel(scalar_representation, idx_m, W1, b1, W2, b2):
    y = _mlp(scalar_representation, W1, b1, W2, b2)
    partials = _sc_segsum(y.reshape(N), idx_m.astype(jnp.int32))
    return partials[0, :M] + partials[1, :M]


# speedup vs baseline: 1.0244x; 1.0244x over previous
import functools

import jax
import jax.numpy as jnp
from jax import lax
from jax.experimental import pallas as pl
from jax.experimental.pallas import tpu as pltpu
from jax.experimental.pallas import tpu_sc as plsc

N = 32768
D = 256
H = 128
M = 16
BLK = 8192

_SC_INFO = plsc.get_sparse_core_info()
NC = _SC_INFO.num_cores
NS = _SC_INFO.num_subcores
NW = NC * NS
CHUNK = N // NW
L = 16


def _mlp_body(x_ref, w1_ref, b1_ref, w2_ref, b2_ref, y_ref):
    x = x_ref[...].astype(jnp.bfloat16)
    h = jnp.dot(x, w1_ref[...].astype(jnp.bfloat16),
                preferred_element_type=jnp.float32)
    h = h + b1_ref[...]
    h = h * jax.nn.sigmoid(h)
    y = jnp.dot(h, w2_ref[...], preferred_element_type=jnp.float32)
    y_ref[...] = y + b2_ref[...]


def _mlp(x, w1, b1, w2, b2):
    return pl.pallas_call(
        _mlp_body,
        grid=(N // BLK,),
        in_specs=[
            pl.BlockSpec((BLK, D), lambda i: (i, 0)),
            pl.BlockSpec((D, H), lambda i: (0, 0)),
            pl.BlockSpec((1, H), lambda i: (0, 0)),
            pl.BlockSpec((H, 1), lambda i: (0, 0)),
            pl.BlockSpec((1, 1), lambda i: (0, 0)),
        ],
        out_specs=pl.BlockSpec((BLK, 1), lambda i: (i, 0)),
        out_shape=jax.ShapeDtypeStruct((N, 1), jnp.float32),
    )(x, w1, b1.reshape(1, H), w2, b2.reshape(1, 1))


CHUNK1 = N // NS


@functools.partial(
    pl.kernel,
    mesh=plsc.VectorSubcoreMesh(core_axis_name="c", subcore_axis_name="s",
                                num_cores=1),
    out_type=jax.ShapeDtypeStruct((128,), jnp.float32),
    scratch_types=[
        pltpu.VMEM((CHUNK1,), jnp.float32),
        pltpu.VMEM((CHUNK1,), jnp.int32),
        pltpu.VMEM((128,), jnp.float32),
        pltpu.VMEM((NS, 128), jnp.float32),
        pltpu.VMEM_SHARED((NS, 128), jnp.float32),
    ],
    compiler_params=pltpu.CompilerParams(needs_layout_passes=False,
                                         skip_device_barrier=True),
)
def _sc_segsum(y_hbm, idx_hbm, out_hbm, y_v, idx_v, acc_v, red_v, shared):
    s = lax.axis_index("s")
    base = s * CHUNK1
    pltpu.sync_copy(y_hbm.at[pl.ds(base, CHUNK1)], y_v)
    pltpu.sync_copy(idx_hbm.at[pl.ds(base, CHUNK1)], idx_v)
    for j in range(128 // L):
        acc_v[pl.ds(j * L, L)] = jnp.zeros((L,), jnp.float32)

    def body(j, carry):
        off = j * L
        v = y_v[pl.ds(off, L)]
        ix = idx_v[pl.ds(off, L)]
        plsc.addupdate_scatter(acc_v, [ix], v)
        return carry

    lax.fori_loop(0, CHUNK1 // L, body, 0, unroll=8)

    pltpu.sync_copy(acc_v, shared.at[s])
    plsc.subcore_barrier()

    @pl.when(s == 0)
    def _reduce():
        pltpu.sync_copy(shared, red_v)
        total = red_v[0, pl.ds(0, M)]
        for j in range(1, NS):
            total = total + red_v[j, pl.ds(0, M)]
        acc_v[pl.ds(0, M)] = total
        pltpu.sync_copy(acc_v, out_hbm)


def kernel(scalar_representation, idx_m, W1, b1, W2, b2):
    y = _mlp(scalar_representation, W1, b1, W2, b2)
    partials = _sc_segsum(y.reshape(N), idx_m.astype(jnp.int32))
    return partials[:M]
